# baseline (speedup 1.0000x reference)
import jax
import jax.numpy as jnp
from jax import lax
from jax.experimental import pallas as pl
from jax.experimental.pallas import tpu as pltpu
from jax.experimental.pallas import tpu_sc as plsc

N = 10000
E = 320000
D = 128
DH = D // 2

NC = 2
NS = 16
NW = NC * NS
CH = 128
NCH = 80
E_PAD = NW * NCH * CH
N_PAD = 10240
RPS = N_PAD // NS
HALF = RPS // 2


def _sc_body(x0_hbm, x1_hbm, src_hbm, dst_hbm, out_hbm,
             src_v, dst_v, rows_a, rows_b, stage_v, zero_v, cnt_v, ones_v,
             acc_sh, cnt_sh, gsem_a, gsem_b):
    c = lax.axis_index("c")
    s = lax.axis_index("s")
    wid = s * NC + c
    base = s * RPS

    zeros16 = jnp.zeros((16,), jnp.float32)
    ones16 = jnp.ones((16,), jnp.float32)

    def zrow(i, carry):
        for v in range(DH // 16):
            zero_v[i, pl.ds(v * 16, 16)] = zeros16
        return carry
    lax.fori_loop(0, HALF, zrow, 0)

    def zcnt(i, carry):
        cnt_v[pl.ds(i * 16, 16)] = zeros16
        return carry
    lax.fori_loop(0, RPS // 16, zcnt, 0)

    for v in range(CH // 16):
        ones_v[pl.ds(v * 16, 16)] = ones16

    pltpu.sync_copy(src_hbm.at[wid], src_v)
    pltpu.sync_copy(dst_hbm.at[pl.ds(s * NC, NC)], dst_v)
    pltpu.sync_copy(cnt_v, cnt_sh.at[pl.ds(base, RPS)])

    for h in range(2):
        xh_hbm = (x0_hbm, x1_hbm)[h]

        pltpu.sync_copy(zero_v, acc_sh.at[pl.ds(base, HALF)])
        pltpu.sync_copy(zero_v, acc_sh.at[pl.ds(base + HALF, HALF)])
        plsc.subcore_barrier()

        def start(j, buf, sem, xh_hbm=xh_hbm):
            pltpu.async_copy(xh_hbm.at[src_v.at[j]], buf, sem)

        def wait(j, buf, sem, xh_hbm=xh_hbm):
            pltpu.make_async_copy(xh_hbm.at[src_v.at[j]], buf, sem).wait()

        start(0, rows_a, gsem_a)
        npair = NCH // 2

        def edge_body(jj, carry, h=h):
            j0 = 2 * jj
            j1 = j0 + 1
            start(j1, rows_b, gsem_b)
            if h == 0:
                pltpu.sync_copy(ones_v, cnt_sh.at[dst_v.at[0, j0]], add=True)
                pltpu.sync_copy(ones_v, cnt_sh.at[dst_v.at[1, j0]], add=True)
            wait(j0, rows_a, gsem_a)
            pltpu.sync_copy(rows_a, acc_sh.at[dst_v.at[c, j0]], add=True)

            @pl.when(jj < npair - 1)
            def _():
                start(j0 + 2, rows_a, gsem_a)

            if h == 0:
                pltpu.sync_copy(ones_v, cnt_sh.at[dst_v.at[0, j1]], add=True)
                pltpu.sync_copy(ones_v, cnt_sh.at[dst_v.at[1, j1]], add=True)
            wait(j1, rows_b, gsem_b)
            pltpu.sync_copy(rows_b, acc_sh.at[dst_v.at[c, j1]], add=True)
            return carry
        lax.fori_loop(0, npair, edge_body, 0)

        plsc.subcore_barrier()

        if h == 0:
            pltpu.sync_copy(cnt_sh.at[pl.ds(base, RPS)], cnt_v)

        for q in range(2):
            rbase = base + q * HALF
            pltpu.sync_copy(acc_sh.at[pl.ds(rbase, HALF)], stage_v)

            def div_body(rr, carry, q=q):
                cvec = jnp.maximum(
                    cnt_v[pl.ds(q * HALF + rr * 16, 16)], 1.0)
                rvec = 1.0 / cvec
                for i in range(16):
                    cval = rvec[i]
                    row = rr * 16 + i
                    for v in range(DH // 16):
                        stage_v[row, pl.ds(v * 16, 16)] = (
                            stage_v[row, pl.ds(v * 16, 16)] * cval)
                return carry
            lax.fori_loop(0, HALF // 16, div_body, 0)

            pltpu.sync_copy(stage_v, out_hbm.at[c, h, pl.ds(rbase, HALF)])

        if h == 0:
            plsc.subcore_barrier()


_sc_aggregate = pl.kernel(
    _sc_body,
    out_type=jax.ShapeDtypeStruct((NC, 2, N_PAD, DH), jnp.float32),
    mesh=plsc.VectorSubcoreMesh(core_axis_name="c", subcore_axis_name="s"),
    compiler_params=pltpu.CompilerParams(use_tc_tiling_on_sc=False),
    scratch_types=[
        pltpu.VMEM((NCH, CH), jnp.int32),
        pltpu.VMEM((NC, NCH, CH), jnp.int32),
        pltpu.VMEM((CH, DH), jnp.float32),
        pltpu.VMEM((CH, DH), jnp.float32),
        pltpu.VMEM((HALF, DH), jnp.float32),
        pltpu.VMEM((HALF, DH), jnp.float32),
        pltpu.VMEM((RPS,), jnp.float32),
        pltpu.VMEM((CH,), jnp.float32),
        pltpu.VMEM_SHARED((N_PAD, DH), jnp.float32),
        pltpu.VMEM_SHARED((N_PAD,), jnp.float32),
        pltpu.SemaphoreType.DMA,
        pltpu.SemaphoreType.DMA,
    ],
)


def _combine_body(m00_ref, m10_ref, m01_ref, m11_ref, x_ref,
                  wl_ref, wr_ref, b_ref, o_ref):
    mean = jnp.concatenate(
        [m00_ref[...] + m10_ref[...], m01_ref[...] + m11_ref[...]], axis=1)
    dn = (((1,), (1,)), ((), ()))
    acc = lax.dot_general(mean, wl_ref[...], dn,
                          preferred_element_type=jnp.float32)
    acc = acc + lax.dot_general(x_ref[...], wr_ref[...], dn,
                                preferred_element_type=jnp.float32)
    acc = acc + b_ref[...]
    o_ref[...] = jnp.maximum(acc, 0.0)


_ROWS_BLK = 1000


def _combine(m00, m10, m01, m11, x, W_l, W_r, b2):
    grid = (N // _ROWS_BLK,)
    half_spec = pl.BlockSpec((_ROWS_BLK, DH), lambda i: (i, 0))
    row_spec = pl.BlockSpec((_ROWS_BLK, D), lambda i: (i, 0))
    full_spec = pl.BlockSpec((D, D), lambda i: (0, 0))
    bias_spec = pl.BlockSpec((1, D), lambda i: (0, 0))
    return pl.pallas_call(
        _combine_body,
        grid=grid,
        in_specs=[half_spec, half_spec, half_spec, half_spec, row_spec,
                  full_spec, full_spec, bias_spec],
        out_specs=row_spec,
        out_shape=jax.ShapeDtypeStruct((N, D), jnp.float32),
    )(m00, m10, m01, m11, x, W_l, W_r, b2)


@jax.jit
def kernel(x, edge_index, W_l, W_r, b):
    src = edge_index[0]
    dst = edge_index[1]
    pad = E_PAD - E
    src_p = jnp.concatenate(
        [src, jnp.zeros((pad,), jnp.int32)]).reshape(NW, NCH, CH)
    dst_p = jnp.concatenate(
        [dst, jnp.full((pad,), N, jnp.int32)]).reshape(NW, NCH, CH)
    x0 = x[:, :DH]
    x1 = x[:, DH:]
    m = _sc_aggregate(x0, x1, src_p, dst_p)
    return _combine(m[0, 0, :N], m[1, 0, :N], m[0, 1, :N], m[1, 1, :N],
                    x, W_l, W_r, b.reshape(1, D))


# speedup vs baseline: 1.1907x; 1.1907x over previous
import jax
import jax.numpy as jnp
from jax import lax
from jax.experimental import pallas as pl
from jax.experimental.pallas import tpu as pltpu
from jax.experimental.pallas import tpu_sc as plsc

N = 10000
E = 320000
D = 128
DH = D // 2

NC = 2
NS = 16
NW = NC * NS
CH = 128
NCH = 79
E_PAD = NW * NCH * CH
N_PAD = 10080
RPS = N_PAD // NS
HALF = RPS // 2
CROWS = 80
CRPS = CROWS // NS
CWIN = 6


def _sc_body(x0_hbm, x1_hbm, src_hbm, dst_hbm, out_hbm,
             src_v, dst_v, rows_v, stage_v, zero_v, cnt_v, cntloc, idx80,
             acc_sh, cnt_sh, gsem):
    c = lax.axis_index("c")
    s = lax.axis_index("s")
    wid = s * NC + c
    base = s * RPS

    zeros16 = jnp.zeros((16,), jnp.float32)
    ones16 = jnp.ones((16,), jnp.float32)
    iota16 = jnp.arange(16, dtype=jnp.int32)

    def zrow(i, carry):
        for v in range(DH // 16):
            zero_v[i, pl.ds(v * 16, 16)] = zeros16
        return carry
    lax.fori_loop(0, HALF, zrow, 0)

    def zcnt(i, carry):
        for v in range(CH // 16):
            cntloc[i, pl.ds(v * 16, 16)] = zeros16
        return carry
    lax.fori_loop(0, CROWS, zcnt, 0)

    for i in range(CROWS // 16):
        idx80[pl.ds(i * 16, 16)] = iota16 + 16 * i

    pltpu.sync_copy(zero_v, acc_sh.at[pl.ds(base, HALF)])
    pltpu.sync_copy(zero_v, acc_sh.at[pl.ds(base + HALF, HALF)])
    pltpu.sync_copy(cntloc.at[pl.ds(0, CRPS)],
                    cnt_sh.at[pl.ds(s * CRPS, CRPS)])

    pltpu.sync_copy(src_hbm.at[wid], src_v)
    pltpu.sync_copy(dst_hbm.at[pl.ds(s * NC, NC)], dst_v)

    for h in range(2):
        xh_hbm = (x0_hbm, x1_hbm)[h]
        if h == 1:
            pltpu.sync_copy(zero_v, acc_sh.at[pl.ds(base, HALF)])
            pltpu.sync_copy(zero_v, acc_sh.at[pl.ds(base + HALF, HALF)])
        plsc.subcore_barrier()

        def edge_body(j, carry, h=h, xh_hbm=xh_hbm):
            pltpu.async_copy(xh_hbm.at[src_v.at[j]], rows_v, gsem).wait()
            pltpu.sync_copy(rows_v, acc_sh.at[dst_v.at[c, j]], add=True)
            if h == 0:
                for k in range(NC):
                    for v in range(CH // 16):
                        dvec = dst_v[k, j, pl.ds(v * 16, 16)]
                        plsc.addupdate_scatter(
                            cntloc, [dvec >> 7, dvec & (CH - 1)], ones16)
            return carry
        lax.fori_loop(0, NCH, edge_body, 0)

        if h == 0:
            pltpu.sync_copy(cntloc, cnt_sh.at[idx80], add=True)

        plsc.subcore_barrier()

        if h == 0:
            pltpu.sync_copy(cnt_sh.at[pl.ds((s * RPS) // CH, CWIN)], cnt_v)

        for q in range(2):
            rbase = base + q * HALF
            pltpu.sync_copy(acc_sh.at[pl.ds(rbase, HALF)], stage_v)

            off = s * RPS - CH * ((s * RPS) // CH)

            def div_body(r, carry, q=q, off=off):
                g = off + q * HALF + r
                cvec = plsc.load_gather(
                    cnt_v, [jnp.full((16,), g >> 7, jnp.int32),
                            jnp.full((16,), g & (CH - 1), jnp.int32)])
                rvec = 1.0 / jnp.maximum(cvec, 1.0)
                for v in range(DH // 16):
                    stage_v[r, pl.ds(v * 16, 16)] = (
                        stage_v[r, pl.ds(v * 16, 16)] * rvec)
                return carry
            lax.fori_loop(0, HALF, div_body, 0)

            pltpu.sync_copy(stage_v, out_hbm.at[c, h, pl.ds(rbase, HALF)])


_sc_aggregate = pl.kernel(
    _sc_body,
    out_type=jax.ShapeDtypeStruct((NC, 2, N_PAD, DH), jnp.float32),
    mesh=plsc.VectorSubcoreMesh(core_axis_name="c", subcore_axis_name="s"),
    compiler_params=pltpu.CompilerParams(use_tc_tiling_on_sc=False,
                                         needs_layout_passes=False),
    scratch_types=[
        pltpu.VMEM((NCH, CH), jnp.int32),
        pltpu.VMEM((NC, NCH, CH), jnp.int32),
        pltpu.VMEM((CH, DH), jnp.float32),
        pltpu.VMEM((HALF, DH), jnp.float32),
        pltpu.VMEM((HALF, DH), jnp.float32),
        pltpu.VMEM((CWIN, CH), jnp.float32),
        pltpu.VMEM((CROWS, CH), jnp.float32),
        pltpu.VMEM((CROWS,), jnp.int32),
        pltpu.VMEM_SHARED((N_PAD, DH), jnp.float32),
        pltpu.VMEM_SHARED((CROWS, CH), jnp.float32),
        pltpu.SemaphoreType.DMA,
    ],
)


def _combine_body(m00_ref, m10_ref, m01_ref, m11_ref, x_ref,
                  wl_ref, wr_ref, b_ref, o_ref):
    mean = jnp.concatenate(
        [m00_ref[...] + m10_ref[...], m01_ref[...] + m11_ref[...]], axis=1)
    dn = (((1,), (1,)), ((), ()))
    acc = lax.dot_general(mean, wl_ref[...], dn,
                          preferred_element_type=jnp.float32)
    acc = acc + lax.dot_general(x_ref[...], wr_ref[...], dn,
                                preferred_element_type=jnp.float32)
    acc = acc + b_ref[...]
    o_ref[...] = jnp.maximum(acc, 0.0)


_ROWS_BLK = 1000


def _combine(m00, m10, m01, m11, x, W_l, W_r, b2):
    grid = (N // _ROWS_BLK,)
    half_spec = pl.BlockSpec((_ROWS_BLK, DH), lambda i: (i, 0))
    row_spec = pl.BlockSpec((_ROWS_BLK, D), lambda i: (i, 0))
    full_spec = pl.BlockSpec((D, D), lambda i: (0, 0))
    bias_spec = pl.BlockSpec((1, D), lambda i: (0, 0))
    return pl.pallas_call(
        _combine_body,
        grid=grid,
        in_specs=[half_spec, half_spec, half_spec, half_spec, row_spec,
                  full_spec, full_spec, bias_spec],
        out_specs=row_spec,
        out_shape=jax.ShapeDtypeStruct((N, D), jnp.float32),
    )(m00, m10, m01, m11, x, W_l, W_r, b2)


@jax.jit
def kernel(x, edge_index, W_l, W_r, b):
    src = edge_index[0]
    dst = edge_index[1]
    pad = E_PAD - E
    src_p = jnp.concatenate(
        [src, jnp.zeros((pad,), jnp.int32)]).reshape(NW, NCH, CH)
    dst_p = jnp.concatenate(
        [dst, jnp.full((pad,), N, jnp.int32)]).reshape(NW, NCH, CH)
    x0 = x[:, :DH]
    x1 = x[:, DH:]
    m = _sc_aggregate(x0, x1, src_p, dst_p)
    return _combine(m[0, 0, :N], m[1, 0, :N], m[0, 1, :N], m[1, 1, :N],
                    x, W_l, W_r, b.reshape(1, D))
